# Initial kernel scaffold; baseline (speedup 1.0000x reference)
import jax
import jax.numpy as jnp
from jax.experimental import pallas as pl


def kernel(x, W1, b1, W2, b2, edge_index):
    raise NotImplementedError("write your pallas kernel here")


# speedup vs baseline: 14.4154x; 14.4154x over previous
import jax
import jax.numpy as jnp
from jax import lax
from jax.experimental import pallas as pl


def _fused_body(x_ref, w1_ref, b1_ref, w2_ref, b2_ref, ei_ref, y_ref):
    ei = ei_ref[...]
    row = ei[0:1, :]
    col = ei[1:2, :]
    n = x_ref.shape[0]
    e = ei.shape[1]
    nodes = lax.broadcasted_iota(jnp.int32, (n, e), 0)
    oh_row = (row == nodes).astype(jnp.float32)
    oh_col = (col == nodes).astype(jnp.float32)
    deg = jnp.sum(oh_col, axis=1, keepdims=True)
    dinv = jnp.where(deg > 0.0, lax.rsqrt(deg), 0.0)
    adj_t = lax.dot_general(
        oh_col * dinv, oh_row * dinv,
        (((1,), (1,)), ((), ())),
        preferred_element_type=jnp.float32,
    )

    w1 = w1_ref[...]
    norm1 = jnp.sqrt(jnp.sum(w1 * w1, axis=0, keepdims=True))
    scale1 = jnp.where(norm1 > 1.0, 1.0 / (norm1 + 1e-7), 1.0)
    xs = x_ref[...] * scale1
    h = lax.dot_general(
        xs, w1, (((1,), (1,)), ((), ())),
        preferred_element_type=jnp.float32,
    )

    agg = lax.dot_general(
        adj_t, h, (((1,), (0,)), ((), ())),
        preferred_element_type=jnp.float32,
    )
    a = agg + b1_ref[...]
    out = jnp.where(a > 0.0, a, jnp.exp(jnp.minimum(a, 0.0)) - 1.0)

    w2 = w2_ref[...]
    norm2 = jnp.sqrt(jnp.sum(w2 * w2, axis=1, keepdims=True))
    scale2 = jnp.where(norm2 > 0.5, 0.5 / (norm2 + 1e-7), 1.0)
    w2n = w2 * scale2
    y_ref[...] = lax.dot_general(
        out, w2n, (((1,), (1,)), ((), ())),
        preferred_element_type=jnp.float32,
    ) + b2_ref[...]


def kernel(x, W1, b1, W2, b2, edge_index):
    n = x.shape[0]
    return pl.pallas_call(
        _fused_body,
        out_shape=jax.ShapeDtypeStruct((n, W2.shape[0]), jnp.float32),
    )(x, W1, b1.reshape(1, -1), W2, b2.reshape(1, -1),
      edge_index.astype(jnp.int32))
